# Initial kernel scaffold; baseline (speedup 1.0000x reference)
import jax
import jax.numpy as jnp
from jax.experimental import pallas as pl


def kernel(x, norm_w, W_r, W_gate, W_up, W_down):
    raise NotImplementedError("write your pallas kernel here")


# speedup vs baseline: 1.1604x; 1.1604x over previous
import functools

import jax
import jax.numpy as jnp
from jax.experimental import pallas as pl
from jax.experimental.pallas import tpu as pltpu


def _moe_body(x_ref, nw_ref, wr_ref, wg_ref, wu_ref, wd_ref,
              out_ref, xn_ref, wsel_ref, *, n_experts, f_tiles):
    e = pl.program_id(0)
    f = pl.program_id(1)

    @pl.when((e == 0) & (f == 0))
    def _prologue():
        x = x_ref[...]
        var = jnp.mean(jnp.square(x), axis=-1, keepdims=True)
        xn = x * jax.lax.rsqrt(var + 1e-6) * nw_ref[...]
        xn_ref[...] = xn.astype(jnp.bfloat16)
        logits = jnp.dot(xn, wr_ref[...], preferred_element_type=jnp.float32)
        probs = jax.nn.softmax(logits, axis=-1)
        lane = jax.lax.broadcasted_iota(jnp.int32, probs.shape, 1)
        m1 = jnp.max(probs, axis=-1, keepdims=True)
        i1 = jnp.min(jnp.where(probs == m1, lane, n_experts), axis=-1,
                     keepdims=True)
        pm = jnp.where(lane == i1, -jnp.inf, probs)
        m2 = jnp.max(pm, axis=-1, keepdims=True)
        i2 = jnp.min(jnp.where(pm == m2, lane, n_experts), axis=-1,
                     keepdims=True)
        sel = (lane == i1) | (lane == i2)
        wsel_ref[...] = jnp.where(sel, probs, 0.0)
        out_ref[...] = x

    xn = xn_ref[...]
    gate = jnp.dot(xn, wg_ref[0], preferred_element_type=jnp.float32)
    up = jnp.dot(xn, wu_ref[0], preferred_element_type=jnp.float32)
    h = gate * jax.lax.logistic(gate) * up
    lane = jax.lax.broadcasted_iota(jnp.int32, wsel_ref.shape, 1)
    w_col = jnp.sum(jnp.where(lane == e, wsel_ref[...], 0.0), axis=-1,
                    keepdims=True)
    hw = (h * w_col).astype(jnp.bfloat16)
    out_ref[...] += jnp.dot(hw, wd_ref[0], preferred_element_type=jnp.float32)


def kernel(x, norm_w, W_r, W_gate, W_up, W_down):
    orig_shape = x.shape
    d = x.shape[-1]
    t = x.size // d
    n_experts, _, d_expert = W_gate.shape
    f_tiles = 2
    fw = d_expert // f_tiles
    x2 = x.reshape(t, d)
    nw2 = norm_w.reshape(1, d)
    wg = W_gate.astype(jnp.bfloat16)
    wu = W_up.astype(jnp.bfloat16)
    wd = W_down.astype(jnp.bfloat16)

    out = pl.pallas_call(
        functools.partial(_moe_body, n_experts=n_experts, f_tiles=f_tiles),
        grid=(n_experts, f_tiles),
        in_specs=[
            pl.BlockSpec((t, d), lambda e, f: (0, 0)),
            pl.BlockSpec((1, d), lambda e, f: (0, 0)),
            pl.BlockSpec((d, n_experts), lambda e, f: (0, 0)),
            pl.BlockSpec((1, d, fw), lambda e, f: (e, 0, f)),
            pl.BlockSpec((1, d, fw), lambda e, f: (e, 0, f)),
            pl.BlockSpec((1, fw, d), lambda e, f: (e, f, 0)),
        ],
        out_specs=pl.BlockSpec((t, d), lambda e, f: (0, 0)),
        out_shape=jax.ShapeDtypeStruct((t, d), jnp.float32),
        scratch_shapes=[
            pltpu.VMEM((t, d), jnp.bfloat16),
            pltpu.VMEM((t, n_experts), jnp.float32),
        ],
    )(x2, nw2, W_r, wg, wu, wd)
    return out.reshape(orig_shape)


# speedup vs baseline: 1.6013x; 1.3801x over previous
import functools

import jax
import jax.numpy as jnp
from jax.experimental import pallas as pl
from jax.experimental.pallas import tpu as pltpu


def _moe_body(x_ref, nw_ref, wr_ref, wg_ref, wu_ref, wd_ref,
              out_ref, xn_ref, wsel_ref, *, n_experts, f_tiles):
    e = pl.program_id(0)
    f = pl.program_id(1)

    @pl.when((e == 0) & (f == 0))
    def _prologue():
        x = x_ref[...]
        var = jnp.mean(jnp.square(x), axis=-1, keepdims=True)
        xn = x * jax.lax.rsqrt(var + 1e-6) * nw_ref[...]
        xn_ref[...] = xn.astype(jnp.bfloat16)
        logits = jnp.dot(xn, wr_ref[...], preferred_element_type=jnp.float32)
        probs = jax.nn.softmax(logits, axis=-1)
        lane = jax.lax.broadcasted_iota(jnp.int32, probs.shape, 1)
        m1 = jnp.max(probs, axis=-1, keepdims=True)
        i1 = jnp.min(jnp.where(probs == m1, lane, n_experts), axis=-1,
                     keepdims=True)
        pm = jnp.where(lane == i1, -jnp.inf, probs)
        m2 = jnp.max(pm, axis=-1, keepdims=True)
        i2 = jnp.min(jnp.where(pm == m2, lane, n_experts), axis=-1,
                     keepdims=True)
        sel = (lane == i1) | (lane == i2)
        wsel_ref[...] = jnp.where(sel, probs, 0.0)
        out_ref[...] = x

    xn = xn_ref[...]
    wg = wg_ref[0].astype(jnp.bfloat16)
    wu = wu_ref[0].astype(jnp.bfloat16)
    wd = wd_ref[0].astype(jnp.bfloat16)
    gate = jnp.dot(xn, wg, preferred_element_type=jnp.float32)
    up = jnp.dot(xn, wu, preferred_element_type=jnp.float32)
    h = gate * jax.lax.logistic(gate) * up
    lane = jax.lax.broadcasted_iota(jnp.int32, wsel_ref.shape, 1)
    w_col = jnp.sum(jnp.where(lane == e, wsel_ref[...], 0.0), axis=-1,
                    keepdims=True)
    hw = (h * w_col).astype(jnp.bfloat16)
    out_ref[...] += jnp.dot(hw, wd, preferred_element_type=jnp.float32)


def kernel(x, norm_w, W_r, W_gate, W_up, W_down):
    orig_shape = x.shape
    d = x.shape[-1]
    t = x.size // d
    n_experts, _, d_expert = W_gate.shape
    f_tiles = 4
    fw = d_expert // f_tiles
    x2 = x.reshape(t, d)
    nw2 = norm_w.reshape(1, d)

    out = pl.pallas_call(
        functools.partial(_moe_body, n_experts=n_experts, f_tiles=f_tiles),
        grid=(n_experts, f_tiles),
        in_specs=[
            pl.BlockSpec((t, d), lambda e, f: (0, 0)),
            pl.BlockSpec((1, d), lambda e, f: (0, 0)),
            pl.BlockSpec((d, n_experts), lambda e, f: (0, 0)),
            pl.BlockSpec((1, d, fw), lambda e, f: (e, 0, f)),
            pl.BlockSpec((1, d, fw), lambda e, f: (e, 0, f)),
            pl.BlockSpec((1, fw, d), lambda e, f: (e, f, 0)),
        ],
        out_specs=pl.BlockSpec((t, d), lambda e, f: (0, 0)),
        out_shape=jax.ShapeDtypeStruct((t, d), jnp.float32),
        scratch_shapes=[
            pltpu.VMEM((t, d), jnp.bfloat16),
            pltpu.VMEM((t, n_experts), jnp.float32),
        ],
    )(x2, nw2, W_r, W_gate, W_up, W_down)
    return out.reshape(orig_shape)
